# baseline (speedup 1.0000x reference)
import jax
import jax.numpy as jnp
from jax.experimental import pallas as pl
from jax.experimental.pallas import tpu as pltpu

B, T, E = 4, 8192, 768
RATIO = 16
H, D = 4, 16
TOPK = 8
G = T // RATIO
HD = H * D

TBLK = 1024
NT = T // TBLK
NGRP = TBLK // RATIO

_EPS = 1e-6
_SCALE = 1.0 / (H * (D ** 0.5))


def _rms_cols(v, m):
    ss = jax.lax.dot_general(v * v, m, (((1,), (0,)), ((), ())),
                             preferred_element_type=jnp.float32,
                             precision=jax.lax.Precision.HIGHEST)
    return v * jax.lax.rsqrt(ss * (1.0 / D) + _EPS)


def _fused(x_ref, w_ref, ape_ref, hm_ref, mask_ref, q_scr, keys_scr):
    p = pl.program_id(1)
    t = pl.program_id(2)

    @pl.when(p == 0)
    def _phase_a():
        x = x_ref[0].astype(jnp.bfloat16)
        proj = jax.lax.dot_general(x, w_ref[...], (((1,), (1,)), ((), ())),
                                   preferred_element_type=jnp.float32)
        q = proj[:, :HD]
        k = proj[:, HD:2 * HD]
        g = proj[:, 2 * HD:]
        g3 = g.reshape(NGRP, RATIO, HD) + ape_ref[...][None]
        g3 = g3 - jnp.max(g3, axis=1, keepdims=True)
        e = jnp.exp(g3)
        wsm = e / jnp.sum(e, axis=1, keepdims=True)
        kk = (k.reshape(NGRP, RATIO, HD) * wsm).sum(axis=1)
        keys_scr[pl.ds(t * NGRP, NGRP), :] = (
            _rms_cols(kk, hm_ref[...]).astype(jnp.bfloat16))
        q_scr[pl.ds(t * TBLK, TBLK), :] = (
            _rms_cols(q, hm_ref[...]).astype(jnp.bfloat16))

    @pl.when(p == 1)
    def _phase_b():
        keys = keys_scr[...]
        neg = jnp.float32(-jnp.inf)
        RB = 64
        for rb in range(TBLK // RB):
            qb = q_scr[pl.ds(t * TBLK + rb * RB, RB), :]
            sc = jax.lax.dot_general(qb, keys, (((1,), (1,)), ((), ())),
                                     preferred_element_type=jnp.float32) * _SCALE
            tg = (t * TBLK + rb * RB
                  + jax.lax.broadcasted_iota(jnp.int32, (RB, G), 0))
            gi = jax.lax.broadcasted_iota(jnp.int32, (RB, G), 1)
            causal = (gi * RATIO + (RATIO - 1)) <= tg
            sc = jnp.where(causal, sc, neg)
            m = jnp.max(sc, axis=-1, keepdims=True)
            for _ in range(TOPK - 1):
                m = jnp.max(jnp.where(sc < m, sc, neg), axis=-1, keepdims=True)
            mask_ref[0, pl.ds(rb * RB, RB), :] = (sc >= m) & (sc > neg)


def _build(interpret=False):
    return pl.pallas_call(
        _fused,
        grid=(B, 2, NT),
        in_specs=[
            pl.BlockSpec((1, TBLK, E),
                         lambda b, p, t: (b, jnp.where(p == 0, t, NT - 1), 0)),
            pl.BlockSpec((3 * HD, E), lambda b, p, t: (0, 0)),
            pl.BlockSpec((RATIO, HD), lambda b, p, t: (0, 0)),
            pl.BlockSpec((HD, HD), lambda b, p, t: (0, 0)),
        ],
        out_specs=pl.BlockSpec((1, TBLK, G),
                               lambda b, p, t: (b, jnp.where(p == 1, t, 0), 0)),
        out_shape=jax.ShapeDtypeStruct((B, T, G), jnp.bool_),
        scratch_shapes=[
            pltpu.VMEM((T, HD), jnp.bfloat16),
            pltpu.VMEM((G, HD), jnp.bfloat16),
        ],
        interpret=interpret,
    )


_FUSED_CALL = _build()


def kernel(x, Wq, Wk, Wg, ape):
    w = jnp.concatenate([Wq, Wk, Wg], axis=0).astype(jnp.bfloat16)
    ape2 = ape.reshape(RATIO, HD)
    head_m = jnp.kron(jnp.eye(H, dtype=jnp.float32),
                      jnp.ones((D, D), dtype=jnp.float32))
    mask = _FUSED_CALL(x, w, ape2, head_m)
    group_ends = jnp.minimum(jnp.arange(RATIO - 1, G * RATIO, RATIO), T - 1)
    return (mask, group_ends)
